# Initial kernel scaffold; baseline (speedup 1.0000x reference)
import functools

import jax
import jax.numpy as jnp
from jax import lax
from jax.experimental import pallas as pl
from jax.experimental.pallas import tpu as pltpu
from jax.experimental.pallas import tpu_sc as plsc

VOCAB = 100000
EMBED = 64
HID = 128
NUM_LABELS = 50
BATCH = 16384
HIST = 200

NC = 2
NS = 16
NW = NC * NS
SPW = BATCH // NW
GROUP = 32
CHUNK = 40
NCHUNK = HIST // CHUNK
COL = EMBED // 16


def _pool_body(x_hbm, emb_hbm, out_hbm, idx_v, rows_a, rows_b, stage_v,
               sem_a, sem_b):
    wid = lax.axis_index("s") * NC + lax.axis_index("c")
    base = wid * SPW

    def stage_idx(g):
        half = lax.rem(g, 2) * GROUP
        pltpu.sync_copy(x_hbm.at[pl.ds(base + g * GROUP, GROUP), :],
                        idx_v.at[pl.ds(half, GROUP), :])

    def fire(s, rows, sem):
        slot = lax.rem(s, 2 * GROUP)
        for c in range(NCHUNK):
            pltpu.async_copy(
                emb_hbm.at[idx_v.at[slot, pl.ds(c * CHUNK, CHUNK)]],
                rows.at[pl.ds(c * CHUNK, CHUNK), :],
                sem)

    def drain(rows, sem):
        pltpu.make_async_copy(emb_hbm.at[pl.ds(0, HIST), :], rows, sem).wait()

    def reduce(rows, s):
        def body(k, carry):
            accs = list(carry)
            r0 = k * 8
            for j in range(8):
                ch = j % 2
                for c in range(COL):
                    accs[ch * COL + c] = (accs[ch * COL + c]
                                          + rows[r0 + j, pl.ds(c * 16, 16)])
            return tuple(accs)

        zero = jnp.zeros((16,), jnp.float32)
        accs = lax.fori_loop(0, HIST // 8, body, (zero,) * (2 * COL))
        slot = lax.rem(s, GROUP)
        for c in range(COL):
            stage_v[slot, pl.ds(c * 16, 16)] = accs[c] + accs[COL + c]

    stage_idx(0)
    fire(0, rows_a, sem_a)
    fire(1, rows_b, sem_b)

    def loop_body(it, carry):
        s0 = it * 2
        s1 = s0 + 1

        drain(rows_a, sem_a)
        reduce(rows_a, s0)

        @pl.when(s0 + 2 < SPW)
        def _():
            @pl.when(lax.rem(s0 + 2, GROUP) == 0)
            def _():
                stage_idx((s0 + 2) // GROUP)
            fire(s0 + 2, rows_a, sem_a)

        drain(rows_b, sem_b)
        reduce(rows_b, s1)

        @pl.when(s1 + 2 < SPW)
        def _():
            fire(s1 + 2, rows_b, sem_b)

        @pl.when(lax.rem(it, GROUP // 2) == GROUP // 2 - 1)
        def _():
            g0 = s1 - (GROUP - 1)
            pltpu.sync_copy(stage_v, out_hbm.at[pl.ds(base + g0, GROUP), :])

        return carry

    lax.fori_loop(0, SPW // 2, loop_body, 0)


_pool = functools.partial(
    pl.kernel,
    out_type=jax.ShapeDtypeStruct((BATCH, EMBED), jnp.float32),
    mesh=plsc.VectorSubcoreMesh(core_axis_name="c", subcore_axis_name="s"),
    scratch_types=[
        pltpu.VMEM((2 * GROUP, HIST), jnp.int32),
        pltpu.VMEM((HIST, EMBED), jnp.float32),
        pltpu.VMEM((HIST, EMBED), jnp.float32),
        pltpu.VMEM((GROUP, EMBED), jnp.float32),
        pltpu.SemaphoreType.DMA,
        pltpu.SemaphoreType.DMA,
    ],
)(_pool_body)


def _mlp_body(sums_ref, w1_ref, b1_ref, w2_ref, b2_ref, out_ref):
    pooled = sums_ref[...] * (1.0 / HIST)
    h = lax.dot_general(pooled, w1_ref[...], (((1,), (1,)), ((), ())),
                        preferred_element_type=jnp.float32)
    h = jnp.maximum(h + b1_ref[...], 0.0)
    out = lax.dot_general(h, w2_ref[...], (((1,), (1,)), ((), ())),
                          preferred_element_type=jnp.float32)
    out_ref[...] = out + b2_ref[...]


_BM = 2048


def _mlp(sums, W1, b1r, W2p, b2p):
    return pl.pallas_call(
        _mlp_body,
        grid=(BATCH // _BM,),
        in_specs=[
            pl.BlockSpec((_BM, EMBED), lambda i: (i, 0)),
            pl.BlockSpec((HID, EMBED), lambda i: (0, 0)),
            pl.BlockSpec((1, HID), lambda i: (0, 0)),
            pl.BlockSpec((HID, HID), lambda i: (0, 0)),
            pl.BlockSpec((1, HID), lambda i: (0, 0)),
        ],
        out_specs=pl.BlockSpec((_BM, HID), lambda i: (i, 0)),
        out_shape=jax.ShapeDtypeStruct((BATCH, HID), jnp.float32),
    )(sums, W1, b1r, W2p, b2p)


def kernel(x, emb, W1, b1, W2, b2):
    x = x.astype(jnp.int32)
    sums = _pool(x, emb)
    b1r = b1.reshape(1, HID)
    W2p = jnp.concatenate([W2, jnp.zeros((HID - NUM_LABELS, HID), W2.dtype)], 0)
    b2p = jnp.concatenate([b2, jnp.zeros((HID - NUM_LABELS,), b2.dtype)], 0)
    logits = _mlp(sums, W1, b1r, W2p, b2p.reshape(1, HID))
    return logits[:, :NUM_LABELS]


# speedup vs baseline: 24.5473x; 24.5473x over previous
import functools

import jax
import jax.numpy as jnp
from jax import lax
from jax.experimental import pallas as pl
from jax.experimental.pallas import tpu as pltpu
from jax.experimental.pallas import tpu_sc as plsc

VOCAB = 100000
EMBED = 64
HID = 128
NUM_LABELS = 50
BATCH = 16384
HIST = 200

NC = 2
NS = 16
NW = NC * NS
SPW = BATCH // NW
GROUP = 32
CHUNK = 40
NCHUNK = HIST // CHUNK
COL = EMBED // 16


def _pool_body(x_hbm, emb_hbm, out_hbm, idx_v, rows_a, rows_b, stage_v,
               sem_a, sem_b):
    wid = lax.axis_index("s") * NC + lax.axis_index("c")
    base = wid * SPW

    def stage_idx(g):
        half = lax.rem(g, 2) * GROUP
        pltpu.sync_copy(x_hbm.at[pl.ds(base + g * GROUP, GROUP), :],
                        idx_v.at[pl.ds(half, GROUP), :])

    def fire(s, rows, sem):
        slot = lax.rem(s, 2 * GROUP)
        for c in range(NCHUNK):
            pltpu.async_copy(
                emb_hbm.at[idx_v.at[slot, pl.ds(c * CHUNK, CHUNK)]],
                rows.at[pl.ds(c * CHUNK, CHUNK), :],
                sem)

    def drain(rows, sem):
        pltpu.make_async_copy(emb_hbm.at[pl.ds(0, HIST), :], rows, sem).wait()

    def reduce(rows, s):
        def body(k, carry):
            accs = list(carry)
            r0 = k * 8
            for j in range(8):
                ch = j % 2
                for c in range(COL):
                    accs[ch * COL + c] = (accs[ch * COL + c]
                                          + rows[r0 + j, pl.ds(c * 16, 16)])
            return tuple(accs)

        zero = jnp.zeros((16,), jnp.float32)
        accs = lax.fori_loop(0, HIST // 8, body, (zero,) * (2 * COL))
        slot = lax.rem(s, GROUP)
        for c in range(COL):
            stage_v[slot, pl.ds(c * 16, 16)] = accs[c] + accs[COL + c]

    stage_idx(0)
    fire(0, rows_a, sem_a)
    fire(1, rows_b, sem_b)

    def loop_body(it, carry):
        s0 = it * 2
        s1 = s0 + 1

        drain(rows_a, sem_a)
        reduce(rows_a, s0)

        @pl.when(s0 + 2 < SPW)
        def _():
            @pl.when(lax.rem(s0 + 2, GROUP) == 0)
            def _():
                stage_idx((s0 + 2) // GROUP)
            fire(s0 + 2, rows_a, sem_a)

        drain(rows_b, sem_b)
        reduce(rows_b, s1)

        @pl.when(s1 + 2 < SPW)
        def _():
            fire(s1 + 2, rows_b, sem_b)

        @pl.when(lax.rem(it, GROUP // 2) == GROUP // 2 - 1)
        def _():
            g0 = s1 - (GROUP - 1)
            pltpu.sync_copy(stage_v, out_hbm.at[pl.ds(base + g0, GROUP), :])

        return carry

    lax.fori_loop(0, SPW // 2, loop_body, 0)


@functools.cache
def _get_pool():
    return functools.partial(
        pl.kernel,
        out_type=jax.ShapeDtypeStruct((BATCH, EMBED), jnp.float32),
        mesh=plsc.VectorSubcoreMesh(core_axis_name="c", subcore_axis_name="s"),
        compiler_params=pltpu.CompilerParams(use_tc_tiling_on_sc=False),
        scratch_types=[
            pltpu.VMEM((2 * GROUP, HIST), jnp.int32),
            pltpu.VMEM((HIST, EMBED), jnp.float32),
            pltpu.VMEM((HIST, EMBED), jnp.float32),
            pltpu.VMEM((GROUP, EMBED), jnp.float32),
            pltpu.SemaphoreType.DMA,
            pltpu.SemaphoreType.DMA,
        ],
    )(_pool_body)


def _mlp_body(sums_ref, w1_ref, b1_ref, w2_ref, b2_ref, out_ref):
    pooled = sums_ref[...] * (1.0 / HIST)
    h = lax.dot_general(pooled, w1_ref[...], (((1,), (1,)), ((), ())),
                        preferred_element_type=jnp.float32)
    h = jnp.maximum(h + b1_ref[...], 0.0)
    out = lax.dot_general(h, w2_ref[...], (((1,), (1,)), ((), ())),
                          preferred_element_type=jnp.float32)
    out_ref[...] = out + b2_ref[...]


_BM = 2048


def _mlp(sums, W1, b1r, W2p, b2p):
    return pl.pallas_call(
        _mlp_body,
        grid=(BATCH // _BM,),
        in_specs=[
            pl.BlockSpec((_BM, EMBED), lambda i: (i, 0)),
            pl.BlockSpec((HID, EMBED), lambda i: (0, 0)),
            pl.BlockSpec((1, HID), lambda i: (0, 0)),
            pl.BlockSpec((HID, HID), lambda i: (0, 0)),
            pl.BlockSpec((1, HID), lambda i: (0, 0)),
        ],
        out_specs=pl.BlockSpec((_BM, HID), lambda i: (i, 0)),
        out_shape=jax.ShapeDtypeStruct((BATCH, HID), jnp.float32),
    )(sums, W1, b1r, W2p, b2p)


def kernel(x, emb, W1, b1, W2, b2):
    x = x.astype(jnp.int32)
    sums = _get_pool()(x, emb)
    b1r = b1.reshape(1, HID)
    W2p = jnp.concatenate([W2, jnp.zeros((HID - NUM_LABELS, HID), W2.dtype)], 0)
    b2p = jnp.concatenate([b2, jnp.zeros((HID - NUM_LABELS,), b2.dtype)], 0)
    logits = _mlp(sums, W1, b1r, W2p, b2p.reshape(1, HID))
    return logits[:, :NUM_LABELS]


# speedup vs baseline: 33.9509x; 1.3831x over previous
import functools

import jax
import jax.numpy as jnp
from jax import lax
from jax.experimental import pallas as pl
from jax.experimental.pallas import tpu as pltpu
from jax.experimental.pallas import tpu_sc as plsc

VOCAB = 100000
EMBED = 64
HID = 128
NUM_LABELS = 50
BATCH = 16384
HIST = 200

NC = 2
NS = 16
NW = NC * NS
SPW = BATCH // NW
GROUP = 32
CHUNKS = ((0, 128), (128, 72))
COL = EMBED // 16
NBUF = 4


def _pool_body(x_hbm, emb_hbm, out_hbm, idx_v, rows_bufs, stage_v, sems):
    wid = lax.axis_index("s") * NC + lax.axis_index("c")
    base = wid * SPW

    def stage_idx(g):
        half = lax.rem(g, 2) * GROUP
        pltpu.sync_copy(x_hbm.at[pl.ds(base + g * GROUP, GROUP), :],
                        idx_v.at[pl.ds(half, GROUP), :])

    def fire(s, rows, sem):
        slot = lax.rem(s, 2 * GROUP)
        for off, num in CHUNKS:
            pltpu.async_copy(
                emb_hbm.at[idx_v.at[slot, pl.ds(off, num)]],
                rows.at[pl.ds(off, num), :],
                sem)

    def drain(rows, sem):
        pltpu.make_async_copy(emb_hbm.at[pl.ds(0, HIST), :], rows, sem).wait()

    def reduce(rows, s):
        def body(k, carry):
            accs = list(carry)
            r0 = k * 8
            for j in range(8):
                ch = j % 2
                for c in range(COL):
                    accs[ch * COL + c] = (accs[ch * COL + c]
                                          + rows[r0 + j, pl.ds(c * 16, 16)])
            return tuple(accs)

        zero = jnp.zeros((16,), jnp.float32)
        accs = lax.fori_loop(0, HIST // 8, body, (zero,) * (2 * COL))
        slot = lax.rem(s, GROUP)
        for c in range(COL):
            stage_v[slot, pl.ds(c * 16, 16)] = accs[c] + accs[COL + c]

    stage_idx(0)
    for j in range(NBUF):
        fire(j, rows_bufs[j], sems[j])

    def loop_body(it, carry):
        for j in range(NBUF):
            s = it * NBUF + j
            drain(rows_bufs[j], sems[j])
            reduce(rows_bufs[j], s)

            @pl.when(s + NBUF < SPW)
            def _(s=s, j=j):
                @pl.when(lax.rem(s + NBUF, GROUP) == 0)
                def _():
                    stage_idx((s + NBUF) // GROUP)
                fire(s + NBUF, rows_bufs[j], sems[j])

        @pl.when(lax.rem(it, GROUP // NBUF) == GROUP // NBUF - 1)
        def _():
            g0 = (it + 1) * NBUF - GROUP
            pltpu.sync_copy(stage_v, out_hbm.at[pl.ds(base + g0, GROUP), :])

        return carry

    lax.fori_loop(0, SPW // NBUF, loop_body, 0)


@functools.cache
def _get_pool():
    return functools.partial(
        pl.kernel,
        out_type=jax.ShapeDtypeStruct((BATCH, EMBED), jnp.float32),
        mesh=plsc.VectorSubcoreMesh(core_axis_name="c", subcore_axis_name="s"),
        compiler_params=pltpu.CompilerParams(use_tc_tiling_on_sc=False),
        scratch_types=[
            pltpu.VMEM((2 * GROUP, HIST), jnp.int32),
            [pltpu.VMEM((HIST, EMBED), jnp.float32) for _ in range(NBUF)],
            pltpu.VMEM((GROUP, EMBED), jnp.float32),
            [pltpu.SemaphoreType.DMA for _ in range(NBUF)],
        ],
    )(_pool_body)


def _mlp_body(sums_ref, w1_ref, b1_ref, w2_ref, b2_ref, out_ref):
    pooled = sums_ref[...] * (1.0 / HIST)
    h = lax.dot_general(pooled, w1_ref[...], (((1,), (1,)), ((), ())),
                        preferred_element_type=jnp.float32)
    h = jnp.maximum(h + b1_ref[...], 0.0)
    out = lax.dot_general(h, w2_ref[...], (((1,), (1,)), ((), ())),
                          preferred_element_type=jnp.float32)
    out_ref[...] = out + b2_ref[...]


_BM = 2048


def _mlp(sums, W1, b1r, W2p, b2p):
    return pl.pallas_call(
        _mlp_body,
        grid=(BATCH // _BM,),
        in_specs=[
            pl.BlockSpec((_BM, EMBED), lambda i: (i, 0)),
            pl.BlockSpec((HID, EMBED), lambda i: (0, 0)),
            pl.BlockSpec((1, HID), lambda i: (0, 0)),
            pl.BlockSpec((HID, HID), lambda i: (0, 0)),
            pl.BlockSpec((1, HID), lambda i: (0, 0)),
        ],
        out_specs=pl.BlockSpec((_BM, HID), lambda i: (i, 0)),
        out_shape=jax.ShapeDtypeStruct((BATCH, HID), jnp.float32),
    )(sums, W1, b1r, W2p, b2p)


def kernel(x, emb, W1, b1, W2, b2):
    x = x.astype(jnp.int32)
    sums = _get_pool()(x, emb)
    b1r = b1.reshape(1, HID)
    W2p = jnp.concatenate([W2, jnp.zeros((HID - NUM_LABELS, HID), W2.dtype)], 0)
    b2p = jnp.concatenate([b2, jnp.zeros((HID - NUM_LABELS,), b2.dtype)], 0)
    logits = _mlp(sums, W1, b1r, W2p, b2p.reshape(1, HID))
    return logits[:, :NUM_LABELS]


# speedup vs baseline: 33.9865x; 1.0010x over previous
import functools

import jax
import jax.numpy as jnp
from jax import lax
from jax.experimental import pallas as pl
from jax.experimental.pallas import tpu as pltpu
from jax.experimental.pallas import tpu_sc as plsc

VOCAB = 100000
EMBED = 64
HID = 128
NUM_LABELS = 50
BATCH = 16384
HIST = 200

NC = 2
NS = 16
NW = NC * NS
SPW = BATCH // NW
GROUP = 32
CHUNKS = ((0, 128), (128, 72))
COL = EMBED // 16
NBUF = 4


def _pool_body(x_hbm, emb_hbm, out_hbm, idx_v, rows_bufs, stage_v, sems):
    wid = lax.axis_index("s") * NC + lax.axis_index("c")
    base = wid * SPW

    def stage_idx(g):
        half = lax.rem(g, 2) * GROUP
        pltpu.sync_copy(x_hbm.at[pl.ds(base + g * GROUP, GROUP), :],
                        idx_v.at[pl.ds(half, GROUP), :])

    def fire(s, rows, sem):
        slot = lax.rem(s, 2 * GROUP)
        for off, num in CHUNKS:
            pltpu.async_copy(
                emb_hbm.at[idx_v.at[slot, pl.ds(off, num)]],
                rows.at[pl.ds(off, num), :],
                sem)

    def drain(rows, sem):
        pltpu.make_async_copy(emb_hbm.at[pl.ds(0, HIST), :], rows, sem).wait()

    def reduce(rows, s):
        def body(k, carry):
            accs = list(carry)
            r0 = k * 8
            for j in range(8):
                ch = j % 2
                for c in range(COL):
                    accs[ch * COL + c] = (accs[ch * COL + c]
                                          + rows[r0 + j, pl.ds(c * 16, 16)])
            return tuple(accs)

        zero = jnp.zeros((16,), jnp.float32)
        accs = lax.fori_loop(0, HIST // 8, body, (zero,) * (2 * COL))
        slot = lax.rem(s, GROUP)
        for c in range(COL):
            stage_v[slot, pl.ds(c * 16, 16)] = accs[c] + accs[COL + c]

    stage_idx(0)
    for j in range(NBUF):
        fire(j, rows_bufs[j], sems[j])

    def loop_body(it, carry):
        for j in range(NBUF):
            s = it * NBUF + j
            drain(rows_bufs[j], sems[j])
            reduce(rows_bufs[j], s)

            @pl.when(s + NBUF < SPW)
            def _(s=s, j=j):
                @pl.when(lax.rem(s + NBUF, GROUP) == 0)
                def _():
                    stage_idx((s + NBUF) // GROUP)
                fire(s + NBUF, rows_bufs[j], sems[j])

        @pl.when(lax.rem(it, GROUP // NBUF) == GROUP // NBUF - 1)
        def _():
            g0 = (it + 1) * NBUF - GROUP
            pltpu.sync_copy(stage_v, out_hbm.at[pl.ds(base + g0, GROUP), :])

        return carry

    lax.fori_loop(0, SPW // NBUF, loop_body, 0)


@functools.cache
def _get_pool():
    return functools.partial(
        pl.kernel,
        out_type=jax.ShapeDtypeStruct((BATCH, EMBED), jnp.float32),
        mesh=plsc.VectorSubcoreMesh(core_axis_name="c", subcore_axis_name="s"),
        compiler_params=pltpu.CompilerParams(use_tc_tiling_on_sc=False),
        scratch_types=[
            pltpu.VMEM((2 * GROUP, HIST), jnp.int32),
            [pltpu.VMEM((HIST, EMBED), jnp.float32) for _ in range(NBUF)],
            pltpu.VMEM((GROUP, EMBED), jnp.float32),
            [pltpu.SemaphoreType.DMA for _ in range(NBUF)],
        ],
    )(_pool_body)


def _mlp_body(sums_ref, w1_ref, b1_ref, w2_ref, b2_ref, out_ref):
    pooled = sums_ref[...] * (1.0 / HIST)
    h = lax.dot_general(pooled, w1_ref[...], (((1,), (1,)), ((), ())),
                        preferred_element_type=jnp.float32)
    h = jnp.maximum(h + b1_ref[...], 0.0)
    out = lax.dot_general(h, w2_ref[...], (((1,), (1,)), ((), ())),
                          preferred_element_type=jnp.float32)
    out_ref[...] = out + b2_ref[...]


_BM = 2048


def _mlp(sums, W1, b1r, W2, b2r):
    return pl.pallas_call(
        _mlp_body,
        grid=(BATCH // _BM,),
        in_specs=[
            pl.BlockSpec((_BM, EMBED), lambda i: (i, 0)),
            pl.BlockSpec((HID, EMBED), lambda i: (0, 0)),
            pl.BlockSpec((1, HID), lambda i: (0, 0)),
            pl.BlockSpec((NUM_LABELS, HID), lambda i: (0, 0)),
            pl.BlockSpec((1, NUM_LABELS), lambda i: (0, 0)),
        ],
        out_specs=pl.BlockSpec((_BM, NUM_LABELS), lambda i: (i, 0)),
        out_shape=jax.ShapeDtypeStruct((BATCH, NUM_LABELS), jnp.float32),
    )(sums, W1, b1r, W2, b2r)


def kernel(x, emb, W1, b1, W2, b2):
    x = x.astype(jnp.int32)
    sums = _get_pool()(x, emb)
    return _mlp(sums, W1, b1.reshape(1, HID), W2, b2.reshape(1, NUM_LABELS))
